# baseline (speedup 1.0000x reference)
import jax
import jax.numpy as jnp
from jax.experimental import pallas as pl

_NOBS, _H, _NPG = 15, 64, 16


def _blockdiag(w, reps, block_r, block_c):
    rows, cols = reps * block_r, reps * block_c
    ri = jax.lax.broadcasted_iota(jnp.int32, (rows, cols), 0) // block_r
    ci = jax.lax.broadcasted_iota(jnp.int32, (rows, cols), 1) // block_c
    return jnp.where(ri == ci, jnp.tile(w, (reps, reps)), 0.0)


def _fused_kernel(x_ref, ws1_ref, wn1_ref, bc1_ref, wp1_ref, wp2_ref,
                  wg1_ref, bg1_ref, wg2_ref, bg2_ref,
                  bp1_ref, bp2_ref, ws2_ref, wn2_ref, bc2_ref,
                  wf1_ref, bf1_ref, wf2_ref, bf2_ref,
                  wf3_ref, bf3_ref, out_ref):
    n = _NPG
    bb = x_ref.shape[0]
    x = x_ref[:]

    bf = jnp.bfloat16
    wsage1 = jnp.concatenate([
        _blockdiag(ws1_ref[:], n, 4, _H),
        jnp.tile(wn1_ref[:], (1, n)),
        jnp.tile(bc1_ref[:], (1, n)),
    ], axis=0).astype(bf)
    wp1d = _blockdiag(wp1_ref[:], n, 4, 4).astype(bf)
    wpp2 = _blockdiag(wp2_ref[:], 2, _H, _H).astype(bf)

    nd64 = jnp.concatenate([jnp.zeros((bb, 4), bf), x[:, 16:].astype(bf)], axis=1)

    m1p = jnp.dot(nd64, wp1d, preferred_element_type=jnp.float32)
    t = jnp.maximum(m1p[:, :32], m1p[:, 32:])
    t = jnp.maximum(t[:, :16], t[:, 16:])
    t = jnp.maximum(t[:, :8], t[:, 8:])
    neigh1 = jnp.maximum(jnp.maximum(t[:, :4], t[:, 4:]) + bp1_ref[:], 0.0)

    a = jnp.concatenate([nd64, neigh1.astype(bf), jnp.ones((bb, 1), bf)], axis=1)

    h1 = jnp.maximum(jnp.dot(a, wsage1,
                             preferred_element_type=jnp.float32), 0.0)
    h1b = h1.astype(bf)
    macc = None
    for j in range(n // 2):
        mp = jnp.dot(h1b[:, 128 * j:128 * (j + 1)], wpp2,
                     preferred_element_type=jnp.float32)
        macc = mp if macc is None else jnp.maximum(macc, mp)

    eyestack = _blockdiag(jnp.ones((1, 1), jnp.float32), _H, 1, 1).astype(bf)
    eyestack = jnp.tile(eyestack, (n, 1))
    neigh2 = jnp.maximum(jnp.maximum(macc[:, :_H], macc[:, _H:]) + bp2_ref[:], 0.0)
    h1mean = jnp.dot(h1b, eyestack,
                     preferred_element_type=jnp.float32) * (1.0 / n)
    obs = (jnp.dot(h1mean, ws2_ref[:], preferred_element_type=jnp.float32)
           + jnp.dot(neigh2, wn2_ref[:], preferred_element_type=jnp.float32)
           + bc2_ref[:])

    g = jnp.maximum(jnp.dot(x[:, :16], wg1_ref[:], preferred_element_type=jnp.float32)
                    + bg1_ref[:], 0.0)
    g = jnp.dot(g, wg2_ref[:], preferred_element_type=jnp.float32) + bg2_ref[:]

    c = jnp.maximum(jnp.dot(g, wf1_ref[:_H], preferred_element_type=jnp.float32)
                    + jnp.dot(obs, wf1_ref[_H:], preferred_element_type=jnp.float32)
                    + bf1_ref[:], 0.0)
    c = jnp.maximum(jnp.dot(c, wf2_ref[:], preferred_element_type=jnp.float32)
                    + bf2_ref[:], 0.0)
    out_ref[:] = jnp.tanh(jnp.dot(c, wf3_ref[:], preferred_element_type=jnp.float32)
                          + bf3_ref[:])


def kernel(X, Wg1, bg1, Wg2, bg2, Wp1, bp1, Ws1, Wn1, bc1, Wp2, bp2, Ws2, Wn2,
           bc2, Wf1, bf1, Wf2, bf2, Wf3, bf3):
    batch = X.shape[0]
    n = _NPG

    full = lambda *s: pl.BlockSpec(s, lambda i: (0,) * len(s))
    specs = [
        pl.BlockSpec((batch, X.shape[1]), lambda i: (0, 0)),
        full(4, _H), full(4, _H), full(1, _H),
        full(4, 4), full(_H, _H),
        full(16, _H), full(1, _H),
        full(_H, _H), full(1, _H),
        full(1, 4), full(1, _H),
        full(_H, _H), full(_H, _H), full(1, _H),
        full(2 * _H, _H), full(1, _H),
        full(_H, _H), full(1, _H),
        full(_H, 8), full(1, 8),
    ]
    out = pl.pallas_call(
        _fused_kernel,
        grid=(1,),
        in_specs=specs,
        out_specs=pl.BlockSpec((batch, 8), lambda i: (0, 0)),
        out_shape=jax.ShapeDtypeStruct((batch, 8), jnp.float32),
    )(X, Ws1, Wn1, bc1.reshape(1, -1), Wp1, Wp2,
      Wg1, bg1.reshape(1, -1), Wg2, bg2.reshape(1, -1),
      bp1.reshape(1, -1), bp2.reshape(1, -1),
      Ws2, Wn2, bc2.reshape(1, -1),
      Wf1, bf1.reshape(1, -1), Wf2, bf2.reshape(1, -1),
      Wf3, bf3.reshape(1, -1))
    return out


# speedup vs baseline: 1.1481x; 1.1481x over previous
import jax
import jax.numpy as jnp
from jax.experimental import pallas as pl

_NOBS, _H, _NPG = 15, 64, 16


def _blockdiag(w, reps, block_r, block_c):
    rows, cols = reps * block_r, reps * block_c
    ri = jax.lax.broadcasted_iota(jnp.int32, (rows, cols), 0) // block_r
    ci = jax.lax.broadcasted_iota(jnp.int32, (rows, cols), 1) // block_c
    return jnp.where(ri == ci, jnp.tile(w, (reps, reps)), 0.0)


def _fused_kernel(x_ref, ws1_ref, wn1_ref, bc1_ref, wp1_ref, wp2_ref,
                  wg1_ref, bg1_ref, wg2_ref, bg2_ref,
                  bp1_ref, bp2_ref, ws2_ref, wn2_ref, bc2_ref,
                  wf1_ref, bf1_ref, wf2_ref, bf2_ref,
                  wf3_ref, bf3_ref, out_ref):
    n = _NPG
    bb = x_ref.shape[0]
    x = x_ref[:]

    bf = jnp.bfloat16
    wsage1 = jnp.concatenate([
        _blockdiag(ws1_ref[:], n, 4, _H),
        jnp.tile(wn1_ref[:], (1, n)),
        jnp.tile(bc1_ref[:], (1, n)),
    ], axis=0).astype(bf)
    wp1d = _blockdiag(wp1_ref[:], n, 4, 4).astype(bf)
    wpp2 = _blockdiag(wp2_ref[:], 2, _H, _H).astype(bf)

    nd64 = jnp.concatenate([jnp.zeros((bb, 4), bf), x[:, 16:].astype(bf)], axis=1)

    m1p = jnp.dot(nd64, wp1d, preferred_element_type=jnp.float32)
    t = jnp.maximum(m1p[:, :32], m1p[:, 32:])
    t = jnp.maximum(t[:, :16], t[:, 16:])
    t = jnp.maximum(t[:, :8], t[:, 8:])
    neigh1 = jnp.maximum(jnp.maximum(t[:, :4], t[:, 4:]) + bp1_ref[:], 0.0)

    a = jnp.concatenate([nd64, neigh1.astype(bf), jnp.ones((bb, 1), bf)], axis=1)

    macc = None
    hsum = None
    for j in range(n // 2):
        hp = jnp.maximum(
            jnp.dot(a, wsage1[:, 128 * j:128 * (j + 1)],
                    preferred_element_type=jnp.float32), 0.0)
        mp = jnp.dot(hp.astype(bf), wpp2,
                     preferred_element_type=jnp.float32)
        macc = mp if macc is None else jnp.maximum(macc, mp)
        hsum = hp if hsum is None else hsum + hp

    neigh2 = jnp.maximum(jnp.maximum(macc[:, :_H], macc[:, _H:]) + bp2_ref[:], 0.0)
    h1mean = (hsum[:, :_H] + hsum[:, _H:]) * (1.0 / n)
    obs = (jnp.dot(h1mean, ws2_ref[:], preferred_element_type=jnp.float32)
           + jnp.dot(neigh2, wn2_ref[:], preferred_element_type=jnp.float32)
           + bc2_ref[:])

    g = jnp.maximum(jnp.dot(x[:, :16], wg1_ref[:], preferred_element_type=jnp.float32)
                    + bg1_ref[:], 0.0)
    g = jnp.dot(g, wg2_ref[:], preferred_element_type=jnp.float32) + bg2_ref[:]

    c = jnp.maximum(jnp.dot(g, wf1_ref[:_H], preferred_element_type=jnp.float32)
                    + jnp.dot(obs, wf1_ref[_H:], preferred_element_type=jnp.float32)
                    + bf1_ref[:], 0.0)
    c = jnp.maximum(jnp.dot(c, wf2_ref[:], preferred_element_type=jnp.float32)
                    + bf2_ref[:], 0.0)
    out_ref[:] = jnp.tanh(jnp.dot(c, wf3_ref[:], preferred_element_type=jnp.float32)
                          + bf3_ref[:])


def kernel(X, Wg1, bg1, Wg2, bg2, Wp1, bp1, Ws1, Wn1, bc1, Wp2, bp2, Ws2, Wn2,
           bc2, Wf1, bf1, Wf2, bf2, Wf3, bf3):
    batch = X.shape[0]
    n = _NPG

    full = lambda *s: pl.BlockSpec(s, lambda i: (0,) * len(s))
    specs = [
        pl.BlockSpec((batch, X.shape[1]), lambda i: (0, 0)),
        full(4, _H), full(4, _H), full(1, _H),
        full(4, 4), full(_H, _H),
        full(16, _H), full(1, _H),
        full(_H, _H), full(1, _H),
        full(1, 4), full(1, _H),
        full(_H, _H), full(_H, _H), full(1, _H),
        full(2 * _H, _H), full(1, _H),
        full(_H, _H), full(1, _H),
        full(_H, 8), full(1, 8),
    ]
    out = pl.pallas_call(
        _fused_kernel,
        grid=(1,),
        in_specs=specs,
        out_specs=pl.BlockSpec((batch, 8), lambda i: (0, 0)),
        out_shape=jax.ShapeDtypeStruct((batch, 8), jnp.float32),
    )(X, Ws1, Wn1, bc1.reshape(1, -1), Wp1, Wp2,
      Wg1, bg1.reshape(1, -1), Wg2, bg2.reshape(1, -1),
      bp1.reshape(1, -1), bp2.reshape(1, -1),
      Ws2, Wn2, bc2.reshape(1, -1),
      Wf1, bf1.reshape(1, -1), Wf2, bf2.reshape(1, -1),
      Wf3, bf3.reshape(1, -1))
    return out
